# baseline (speedup 1.0000x reference)
import functools

import jax
import jax.numpy as jnp
import numpy as np
from jax.experimental import pallas as pl
from jax.experimental.pallas import tpu as pltpu

_OUT_H, _OUT_W = 256, 128
_MEANS = (103.94, 116.78, 123.68)
_STD = (57.38, 57.12, 58.4)


def _interp_matrix(out_size: int, in_size: int) -> np.ndarray:
    scale = in_size / out_size
    o = np.arange(out_size, dtype=np.float32)
    src = np.maximum((o + 0.5) * scale - 0.5, 0.0)
    x0 = np.clip(np.floor(src).astype(np.int32), 0, in_size - 1)
    x1 = np.minimum(x0 + 1, in_size - 1)
    lam = (src - x0.astype(np.float32)).astype(np.float32)
    rows = np.arange(out_size)
    m = np.zeros((out_size, in_size), np.float32)
    np.add.at(m, (rows, x0), 1.0 - lam)
    np.add.at(m, (rows, x1), lam)
    return m


def _halfplane_kernel(xa_ref, xb_ref, xc_ref, ww_ref, wh_ref, o_ref, acc_ref,
                      *, KH, OFFS):
    k = pl.program_id(1)

    @pl.when(k == 0)
    def _init():
        acc_ref[...] = jnp.zeros_like(acc_ref)

    wh = wh_ref[...]
    for c, x_ref in enumerate((xa_ref, xb_ref, xc_ref)):
        tmp = jnp.dot(x_ref[...], ww_ref[c],
                      preferred_element_type=jnp.float32)
        acc_ref[c, :, :] += jnp.dot(wh, tmp,
                                    preferred_element_type=jnp.float32)

    @pl.when(k == KH - 1)
    def _finalize():
        for c in range(3):
            o_ref[c, :, :] = acc_ref[c, :, :] - OFFS[c]


def kernel(img: jnp.ndarray) -> jnp.ndarray:
    N, H, W, C = img.shape
    assert C == 3, "expects 3-channel input"

    x_pl = jnp.transpose(img.astype(jnp.float32), (0, 3, 1, 2))

    KH = 2 if (H % 16 == 0) else 1
    TH = H // KH

    ww = _interp_matrix(_OUT_W, W).T
    ww_c = np.stack([ww / _STD[2 - c] for c in range(3)])
    wh = _interp_matrix(_OUT_H, H)
    wh_kh = wh.reshape(_OUT_H, KH, TH).transpose(1, 0, 2)
    offs = tuple(float(_MEANS[2 - c] / _STD[2 - c]) for c in range(3))

    kern = functools.partial(_halfplane_kernel, KH=KH, OFFS=offs)
    out_shape = jax.ShapeDtypeStruct((N, 3, _OUT_H, _OUT_W), jnp.float32)
    return pl.pallas_call(
        kern,
        out_shape=out_shape,
        grid=(N, KH),
        in_specs=[
            pl.BlockSpec((None, None, TH, W), lambda n, k: (n, 2, k, 0)),
            pl.BlockSpec((None, None, TH, W), lambda n, k: (n, 1, k, 0)),
            pl.BlockSpec((None, None, TH, W), lambda n, k: (n, 0, k, 0)),
            pl.BlockSpec((3, W, _OUT_W), lambda n, k: (0, 0, 0)),
            pl.BlockSpec((None, _OUT_H, TH), lambda n, k: (k, 0, 0)),
        ],
        out_specs=pl.BlockSpec((None, 3, _OUT_H, _OUT_W),
                               lambda n, k: (n, 0, 0, 0)),
        scratch_shapes=[pltpu.VMEM((3, _OUT_H, _OUT_W), jnp.float32)],
        compiler_params=pltpu.CompilerParams(
            dimension_semantics=("parallel", "arbitrary"),
            vmem_limit_bytes=48 * 1024 * 1024,
        ),
    )(x_pl, x_pl, x_pl,
      jnp.asarray(ww_c.astype(jnp.bfloat16)),
      jnp.asarray(wh_kh.astype(jnp.bfloat16)))


# speedup vs baseline: 1.1738x; 1.1738x over previous
import functools

import jax
import jax.numpy as jnp
import numpy as np
from jax.experimental import pallas as pl
from jax.experimental.pallas import tpu as pltpu

_OUT_H, _OUT_W = 256, 128
_MEANS = (103.94, 116.78, 123.68)
_STD = (57.38, 57.12, 58.4)


def _interp_matrix(out_size: int, in_size: int) -> np.ndarray:
    scale = in_size / out_size
    o = np.arange(out_size, dtype=np.float32)
    src = np.maximum((o + 0.5) * scale - 0.5, 0.0)
    x0 = np.clip(np.floor(src).astype(np.int32), 0, in_size - 1)
    x1 = np.minimum(x0 + 1, in_size - 1)
    lam = (src - x0.astype(np.float32)).astype(np.float32)
    rows = np.arange(out_size)
    m = np.zeros((out_size, in_size), np.float32)
    np.add.at(m, (rows, x0), 1.0 - lam)
    np.add.at(m, (rows, x1), lam)
    return m


def _batch_kernel(xa_ref, xb_ref, xc_ref, ww_ref, wh_ref, o_ref, *, OFFS):
    wh = wh_ref[...]
    for c, x_ref in enumerate((xa_ref, xb_ref, xc_ref)):
        x = x_ref[...].astype(jnp.bfloat16)
        tmp = jnp.dot(x, ww_ref[c],
                      preferred_element_type=jnp.float32)
        out = jnp.dot(wh, tmp.astype(jnp.bfloat16),
                      preferred_element_type=jnp.float32)
        o_ref[c, :, :] = out - OFFS[c]


def kernel(img: jnp.ndarray) -> jnp.ndarray:
    N, H, W, C = img.shape
    assert C == 3, "expects 3-channel input"

    x_pl = jnp.transpose(img.astype(jnp.float32), (0, 3, 1, 2))

    ww = _interp_matrix(_OUT_W, W).T
    ww_c = np.stack([ww / _STD[2 - c] for c in range(3)])
    wh = _interp_matrix(_OUT_H, H)
    offs = tuple(float(_MEANS[2 - c] / _STD[2 - c]) for c in range(3))

    kern = functools.partial(_batch_kernel, OFFS=offs)
    out_shape = jax.ShapeDtypeStruct((N, 3, _OUT_H, _OUT_W), jnp.float32)
    return pl.pallas_call(
        kern,
        out_shape=out_shape,
        grid=(N,),
        in_specs=[
            pl.BlockSpec((None, None, H, W), lambda n: (n, 2, 0, 0)),
            pl.BlockSpec((None, None, H, W), lambda n: (n, 1, 0, 0)),
            pl.BlockSpec((None, None, H, W), lambda n: (n, 0, 0, 0)),
            pl.BlockSpec((3, W, _OUT_W), lambda n: (0, 0, 0)),
            pl.BlockSpec((_OUT_H, H), lambda n: (0, 0)),
        ],
        out_specs=pl.BlockSpec((None, 3, _OUT_H, _OUT_W),
                               lambda n: (n, 0, 0, 0)),
        compiler_params=pltpu.CompilerParams(
            dimension_semantics=("parallel",),
            vmem_limit_bytes=48 * 1024 * 1024,
        ),
    )(x_pl, x_pl, x_pl,
      jnp.asarray(ww_c.astype(jnp.bfloat16)),
      jnp.asarray(wh.astype(jnp.bfloat16)))
